# baseline (speedup 1.0000x reference)
import functools

import jax
import jax.numpy as jnp
from jax import lax
from jax.experimental import pallas as pl
from jax.experimental.pallas import tpu as pltpu
from jax.experimental.pallas import tpu_sc as plsc

_B, _S, _D = 4, 8192, 768
_NC = 2
_ROWS_C = _S // _NC
_CH = 512
_CHUNKS = _ROWS_C // _CH

_mesh = plsc.ScalarSubcoreMesh(axis_name="c", num_cores=_NC)


@functools.partial(
    pl.kernel,
    mesh=_mesh,
    out_type=jax.ShapeDtypeStruct((_B, _S, _D), jnp.float32),
    scratch_types=[pltpu.VMEM_SHARED((2, _CH, _D), jnp.float32)]
    + [pltpu.SemaphoreType.DMA] * 4,
)
def _scs_broadcast_copy(table_hbm, out_hbm, buf, *sems):
    rsems, wsems = sems[:2], sems[2:]
    cid = lax.axis_index("c")
    base = cid * _ROWS_C
    writes = [[], []]
    reads = [None, None]

    for i in range(2):
        reads[i] = pltpu.async_copy(
            table_hbm.at[pl.ds(base + i * _CH, _CH)], buf.at[i], rsems[i]
        )
    for i in range(_CHUNKS):
        sl = i % 2
        reads[sl].wait()
        r0 = base + i * _CH
        for b in range(_B):
            writes[sl].append(
                pltpu.async_copy(buf.at[sl], out_hbm.at[b, pl.ds(r0, _CH)], wsems[sl])
            )
        nxt = i + 2
        if nxt < _CHUNKS:
            for w in writes[sl]:
                w.wait()
            writes[sl] = []
            reads[sl] = pltpu.async_copy(
                table_hbm.at[pl.ds(base + nxt * _CH, _CH)], buf.at[sl], rsems[sl]
            )
    for sl in range(2):
        for w in writes[sl]:
            w.wait()


def kernel(x, pos_embed_weight):
    del x
    return _scs_broadcast_copy(pos_embed_weight)


# speedup vs baseline: 1.4310x; 1.4310x over previous
import functools

import jax
import jax.numpy as jnp
from jax import lax
from jax.experimental import pallas as pl
from jax.experimental.pallas import tpu as pltpu
from jax.experimental.pallas import tpu_sc as plsc
from jax._src.pallas import mpmd

_B, _S, _D = 4, 8192, 768
_NC, _NS = 2, 16
_NW = _NC * _NS

_S_TEC = 4608
_TEC_CH = 72
_TEC_CHUNKS = (_S_TEC // _NW) // _TEC_CH

_S_SCS = _S - _S_TEC
_SCS_CH = 448
_SCS_CHUNKS = (_S_SCS // _NC) // _SCS_CH

_mesh_v = plsc.VectorSubcoreMesh(core_axis_name="c", subcore_axis_name="s")
_mesh_s = plsc.ScalarSubcoreMesh(axis_name="c", num_cores=_NC)


def _ring_copy(table_hbm, out_hbm, buf, rsems, wsems, base, ch, n_chunks):
    writes = [[], []]
    reads = [None, None]
    for i in range(min(2, n_chunks)):
        reads[i] = pltpu.async_copy(
            table_hbm.at[pl.ds(base + i * ch, ch)], buf.at[i], rsems[i]
        )
    for i in range(n_chunks):
        sl = i % 2
        reads[sl].wait()
        r0 = base + i * ch
        for b in range(_B):
            writes[sl].append(
                pltpu.async_copy(buf.at[sl], out_hbm.at[b, pl.ds(r0, ch)], wsems[sl])
            )
        nxt = i + 2
        if nxt < n_chunks:
            for w in writes[sl]:
                w.wait()
            writes[sl] = []
            reads[sl] = pltpu.async_copy(
                table_hbm.at[pl.ds(base + nxt * ch, ch)], buf.at[sl], rsems[sl]
            )
    for sl in range(2):
        for w in writes[sl]:
            w.wait()


def _tec_fn(table_hbm, out_hbm):
    wid = lax.axis_index("s") * _NC + lax.axis_index("c")
    base = wid * (_S_TEC // _NW)

    def body(buf, r0, r1, w0, w1):
        _ring_copy(table_hbm, out_hbm, buf, (r0, r1), (w0, w1),
                   base, _TEC_CH, _TEC_CHUNKS)

    pl.run_scoped(
        body,
        pltpu.VMEM((2, _TEC_CH, _D), jnp.float32),
        *([pltpu.SemaphoreType.DMA] * 4),
    )


def _scs_fn(table_hbm, out_hbm):
    cid = lax.axis_index("c")
    base = _S_TEC + cid * (_S_SCS // _NC)

    def body(buf, r0, r1, w0, w1):
        _ring_copy(table_hbm, out_hbm, buf, (r0, r1), (w0, w1),
                   base, _SCS_CH, _SCS_CHUNKS)

    pl.run_scoped(
        body,
        pltpu.VMEM_SHARED((2, _SCS_CH, _D), jnp.float32),
        *([pltpu.SemaphoreType.DMA] * 4),
    )


_combined = mpmd.mpmd_map(
    [(_mesh_s, _scs_fn), (_mesh_v, _tec_fn)],
    out_types=jax.ShapeDtypeStruct((_B, _S, _D), jnp.float32),
)


def kernel(x, pos_embed_weight):
    del x
    return _combined(pos_embed_weight)
